# baseline (speedup 1.0000x reference)
import functools

import jax
import jax.numpy as jnp
from jax import lax
from jax.experimental import pallas as pl
from jax.experimental.pallas import tpu as pltpu
from jax.experimental.pallas import tpu_sc as plsc

N_SRC, N_MID, N_DST = 10000, 5000, 2000
D = 128
VOCAB_ID = 100000
P = 2000

NC, NS = 2, 16
NW = NC * NS
CH = 128

NPR = 10240
NP_MID = 5120
NP_DST = 2048
E0P = 80 * 4096
E1P = 40 * 4096
PP = 4096

_f32 = jnp.float32
_i32 = jnp.int32


def _mesh():
    return plsc.VectorSubcoreMesh(core_axis_name="c", subcore_axis_name="s",
                                  num_cores=NC, num_subcores=NS)


def _zero_vec(ref, n_rows, n_cols):
    @pl.loop(0, n_rows)
    def _(r):
        for j in range(n_cols // 16):
            ref[r, pl.ds(j * 16, 16)] = jnp.zeros((16,), _f32)


def _acc_add(acc_v, buf):
    @pl.loop(0, CH)
    def _(r):
        for j in range(D // 16):
            sl = pl.ds(j * 16, 16)
            acc_v[r, sl] = acc_v[r, sl] + buf[r, sl]


def _gather_sum_body(tflat, idtab, g_part, fc, ids_in, h_out,
                     idx_a, idx_b, buf_a, buf_b, acc_v, sem_a, sem_b):
    c = lax.axis_index("c")
    s = lax.axis_index("s")
    cpc = NPR // CH // NC

    def chunk(off):
        pltpu.sync_copy(g_part.at[pl.ds(off, CH)], acc_v)
        pltpu.sync_copy(ids_in.at[pl.ds(off, CH)], idx_a)
        pltpu.async_copy(idtab.at[idx_a], buf_a, sem_a)
        for m in range(9):
            nidx, nbuf, nsem = (idx_b, buf_b, sem_b) if m % 2 == 0 else \
                               (idx_a, buf_a, sem_a)
            cbuf, csem = (buf_a, sem_a) if m % 2 == 0 else (buf_b, sem_b)
            pltpu.sync_copy(fc.at[pl.ds(m * NPR + off, CH)], nidx)
            for j in range(CH // 16):
                sl = pl.ds(j * 16, 16)
                nidx[sl] = nidx[sl] + (m * 100)
            pltpu.async_copy(tflat.at[nidx], nbuf, nsem)
            pltpu.make_async_copy(idtab.at[idx_a], cbuf, csem).wait()
            _acc_add(acc_v, cbuf)
        lbuf, lsem = (buf_b, sem_b) if 9 % 2 == 1 else (buf_a, sem_a)
        pltpu.make_async_copy(idtab.at[idx_a], lbuf, lsem).wait()
        _acc_add(acc_v, lbuf)
        pltpu.sync_copy(acc_v, h_out.at[pl.ds(off, CH)])

    del cpc
    for kk in range(3):
        @pl.when(c == 0)
        def _():
            chunk((s + NS * kk) * CH)

    for kk in range(2):
        @pl.when(c == 1)
        def _():
            chunk((3 * NS + kk * NS + s) * CH)


def _gather_sum(tflat, idtab, g_part, fc, ids_in):
    kfn = pl.kernel(
        _gather_sum_body,
        out_type=jax.ShapeDtypeStruct((NPR, D), _f32),
        mesh=_mesh(),
        scratch_types=[
            pltpu.VMEM((CH,), _i32),
            pltpu.VMEM((CH,), _i32),
            pltpu.VMEM((CH, D), _f32),
            pltpu.VMEM((CH, D), _f32),
            pltpu.VMEM((CH, D), _f32),
            pltpu.SemaphoreType.DMA,
            pltpu.SemaphoreType.DMA,
        ],
    )
    return kfn(tflat, idtab, g_part, fc, ids_in)


def _edge_agg_body(n_hbm, src, dst, w, agg_out, ws_out,
                   si_v, di_v, w_v, g_a, g_b, z1_v,
                   sem_a, sem_b, sem_c, sem_d, agg_s, ws_s,
                   *, n_pad, e_pad, ca, cb_n):
    c = lax.axis_index("c")
    s = lax.axis_index("s")
    rows_per_tile = n_pad // NS

    _zero_vec(g_a, CH, D)
    for j in range(CH // 16):
        z1_v[pl.ds(j * 16, 16)] = jnp.zeros((16,), _f32)
    r0 = s * rows_per_tile
    off = 0
    while off < rows_per_tile:
        n = min(CH, rows_per_tile - off)
        pltpu.sync_copy(g_a.at[pl.ds(0, n)], agg_s.at[pl.ds(r0 + off, n)])
        pltpu.sync_copy(z1_v.at[pl.ds(0, n)], ws_s.at[pl.ds(r0 + off, n)])
        off += n
    plsc.subcore_barrier()

    def scale(buf, k):
        @pl.loop(0, CH // 16)
        def _(g):
            wv = w_v[k, pl.ds(g * 16, 16)]
            for l in range(16):
                spl = jnp.broadcast_to(wv[l:l + 1], (16,))
                e = g * 16 + l
                for j in range(D // 16):
                    sl = pl.ds(j * 16, 16)
                    buf[e, sl] = buf[e, sl] * spl

    def emit(buf, k, sem):
        pltpu.async_copy(buf, agg_s.at[di_v.at[k]], sem, add=True)
        pltpu.async_copy(w_v.at[k], ws_s.at[di_v.at[k]], sem, add=True)

    def drain(buf, sem):
        pltpu.make_async_copy(buf, agg_s.at[di_v.at[0]], sem).wait()
        pltpu.make_async_copy(w_v.at[0], ws_s.at[di_v.at[0]], sem).wait()

    def run(n_chunks, cb):
        pltpu.sync_copy(src.at[pl.ds(cb, n_chunks)],
                        si_v.at[pl.ds(0, n_chunks)])
        pltpu.sync_copy(dst.at[pl.ds(cb, n_chunks)],
                        di_v.at[pl.ds(0, n_chunks)])
        pltpu.sync_copy(w.at[pl.ds(cb, n_chunks)],
                        w_v.at[pl.ds(0, n_chunks)])

        pltpu.async_copy(n_hbm.at[si_v.at[0]], g_a, sem_a)

        @pl.loop(0, n_chunks // 2)
        def _(p):
            k0 = 2 * p
            pltpu.make_async_copy(n_hbm.at[si_v.at[0]], g_a, sem_a).wait()
            pltpu.async_copy(n_hbm.at[si_v.at[k0 + 1]], g_b, sem_b)
            scale(g_a, k0)
            emit(g_a, k0, sem_c)

            pltpu.make_async_copy(n_hbm.at[si_v.at[0]], g_b, sem_b).wait()
            drain(g_a, sem_c)

            @pl.when(k0 + 2 < n_chunks)
            def _():
                pltpu.async_copy(n_hbm.at[si_v.at[k0 + 2]], g_a, sem_a)
            scale(g_b, k0 + 1)
            emit(g_b, k0 + 1, sem_d)
            drain(g_b, sem_d)

    @pl.when(c == 0)
    def _():
        run(ca, s * ca)

    if cb_n > 0:
        @pl.when(c == 1)
        def _():
            run(cb_n, NS * ca + s * cb_n)

    plsc.subcore_barrier()
    off = 0
    while off < rows_per_tile:
        n = min(CH, rows_per_tile - off)
        pltpu.sync_copy(agg_s.at[pl.ds(r0 + off, n)], g_a.at[pl.ds(0, n)])
        pltpu.sync_copy(g_a.at[pl.ds(0, n)], agg_out.at[c, pl.ds(r0 + off, n)])
        pltpu.sync_copy(ws_s.at[pl.ds(r0 + off, n)], z1_v.at[pl.ds(0, n)])
        pltpu.sync_copy(z1_v.at[pl.ds(0, n)],
                        ws_out.at[pl.ds(c * n_pad + r0 + off, n)])
        off += n


def _edge_agg(n_hbm, src2d, dst2d, w2d, n_pad, e_pad, frac0=0.5):
    pair_chunks = e_pad // NS // CH
    ca = max(8, 8 * int(round(pair_chunks * frac0 / 8)))
    ca = min(ca, pair_chunks)
    cb_n = pair_chunks - ca
    body = functools.partial(_edge_agg_body, n_pad=n_pad, e_pad=e_pad,
                             ca=ca, cb_n=cb_n)
    kfn = pl.kernel(
        body,
        out_type=(jax.ShapeDtypeStruct((NC, n_pad, D), _f32),
                  jax.ShapeDtypeStruct((NC * n_pad,), _f32)),
        mesh=_mesh(),
        scratch_types=[
            pltpu.VMEM((max(ca, cb_n), CH), _i32),
            pltpu.VMEM((max(ca, cb_n), CH), _i32),
            pltpu.VMEM((max(ca, cb_n), CH), _f32),
            pltpu.VMEM((CH, D), _f32),
            pltpu.VMEM((CH, D), _f32),
            pltpu.VMEM((CH,), _f32),
            pltpu.SemaphoreType.DMA,
            pltpu.SemaphoreType.DMA,
            pltpu.SemaphoreType.DMA,
            pltpu.SemaphoreType.DMA,
            pltpu.VMEM_SHARED((n_pad, D), _f32),
            pltpu.VMEM_SHARED((n_pad,), _f32),
        ],
    )
    return kfn(n_hbm, src2d, dst2d, w2d)


def _score_body(h_item, ue, ve, nid_h, bias_h, ru_o, rv_o, bu_o, bv_o,
                ui_v, vi_v, ru_v, rv_v, nu_v, bu_v, sem):
    c = lax.axis_index("c")
    s = lax.axis_index("s")
    wid = c * NS + s
    base = wid * (PP // NW)
    pltpu.sync_copy(ue.at[pl.ds(base, CH)], ui_v)
    pltpu.sync_copy(ve.at[pl.ds(base, CH)], vi_v)
    pltpu.async_copy(h_item.at[ui_v], ru_v, sem).wait()
    pltpu.sync_copy(ru_v, ru_o.at[pl.ds(base, CH)])
    pltpu.async_copy(h_item.at[vi_v], rv_v, sem).wait()
    pltpu.sync_copy(rv_v, rv_o.at[pl.ds(base, CH)])
    pltpu.async_copy(nid_h.at[ui_v], nu_v, sem).wait()
    pltpu.async_copy(bias_h.at[nu_v], bu_v, sem).wait()
    pltpu.sync_copy(bu_v, bu_o.at[pl.ds(base, CH)])
    pltpu.async_copy(nid_h.at[vi_v], nu_v, sem).wait()
    pltpu.async_copy(bias_h.at[nu_v], bu_v, sem).wait()
    pltpu.sync_copy(bu_v, bv_o.at[pl.ds(base, CH)])


def _score(h_item, ue, ve, nid_h, bias_h):
    kfn = pl.kernel(
        _score_body,
        out_type=(jax.ShapeDtypeStruct((PP, D), _f32),
                  jax.ShapeDtypeStruct((PP, D), _f32),
                  jax.ShapeDtypeStruct((PP,), _f32),
                  jax.ShapeDtypeStruct((PP,), _f32)),
        mesh=_mesh(),
        scratch_types=[
            pltpu.VMEM((CH,), _i32),
            pltpu.VMEM((CH,), _i32),
            pltpu.VMEM((CH, D), _f32),
            pltpu.VMEM((CH, D), _f32),
            pltpu.VMEM((CH,), _i32),
            pltpu.VMEM((CH,), _f32),
            pltpu.SemaphoreType.DMA,
        ],
    )
    return kfn(h_item, ue, ve, nid_h, bias_h)


def _dot_t(x, w):
    return lax.dot_general(x, w, (((1,), (1,)), ((), ())),
                           preferred_element_type=_f32)


def _prep_body(emb_ref, wfc_ref, genre_ref, bfc_ref, t_ref, g_ref):
    for m in range(9):
        wm = wfc_ref[:, 16 * m:16 * (m + 1)]
        t_ref[m] = _dot_t(emb_ref[m], wm)
    wg = wfc_ref[:, 144:164]
    g_ref[...] = _dot_t(genre_ref[...], wg) + bfc_ref[...]


def _dense1_body(h_ref, q_ref, bq_ref, n_ref):
    n_ref[...] = jnp.maximum(_dot_t(h_ref[...], q_ref[...]) + bq_ref[...], 0.0)


def _combine_body(agg_ref, ws_ref, hd_ref, w_ref, bw_ref, q_ref, bq_ref,
                  h1_ref, n1_ref, *, n_dst):
    p = agg_ref[0, pl.ds(0, n_dst)] + agg_ref[1, pl.ds(0, n_dst)]
    ws = jnp.clip(ws_ref[0, pl.ds(0, n_dst)] + ws_ref[1, pl.ds(0, n_dst)],
                  1.0, None)
    x = p / ws[:, None]
    hd = hd_ref[pl.ds(0, n_dst)]
    wa = w_ref[:, 0:D]
    wb = w_ref[:, D:2 * D]
    z = jnp.maximum(_dot_t(x, wa) + _dot_t(hd, wb) + bw_ref[...], 0.0)
    zn = jnp.sqrt(jnp.sum(z * z, axis=1, keepdims=True))
    zn = jnp.where(zn == 0.0, 1.0, zn)
    h1 = z / zn
    h1_ref[...] = h1
    if n1_ref is not None:
        n1_ref[...] = jnp.maximum(_dot_t(h1, q_ref[...]) + bq_ref[...], 0.0)


def _combine3_body(agg_ref, ws_ref, hd_ref, w_ref, bw_ref, hp_ref, hi_ref):
    p = agg_ref[0, pl.ds(0, N_DST)] + agg_ref[1, pl.ds(0, N_DST)]
    ws = jnp.clip(ws_ref[0, pl.ds(0, N_DST)] + ws_ref[1, pl.ds(0, N_DST)],
                  1.0, None)
    x = p / ws[:, None]
    hd = hd_ref[pl.ds(0, N_DST)]
    wa = w_ref[:, 0:D]
    wb = w_ref[:, D:2 * D]
    z = jnp.maximum(_dot_t(x, wa) + _dot_t(hd, wb) + bw_ref[...], 0.0)
    zn = jnp.sqrt(jnp.sum(z * z, axis=1, keepdims=True))
    zn = jnp.where(zn == 0.0, 1.0, zn)
    hi_ref[...] = hp_ref[pl.ds(0, N_DST)] + z / zn


def _finish_body(ru_ref, rv_ref, bu_ref, bv_ref, loss_ref, auc_ref):
    s = (jnp.sum(ru_ref[...] * rv_ref[...], axis=1)
         + bu_ref[...] + bv_ref[...])
    half = PP // 2
    pos = s[0:half]
    neg = s[half:PP]
    loss_ref[...] = jnp.maximum(neg - pos + 1.0, 0.0)
    valid = lax.broadcasted_iota(_i32, (half,), 0) < P
    pv = jnp.where(valid, pos, -jnp.inf)
    nv = jnp.where(valid, neg, jnp.inf)
    gt = jnp.sum((pv[:, None] > nv[None, :]).astype(_f32))
    eq = jnp.sum((pv[:, None] == nv[None, :]).astype(_f32))
    auc_ref[...] = jnp.broadcast_to((gt + 0.5 * eq) / (float(P) * float(P)),
                                    (1, 1))


def kernel(feat_cat, ids, genre, src0, dst0, w0, src1, dst1, w1, pos_edges,
           neg_edges, nid, emb_feats, id_table, Wfc, bfc, Q0, bq0, W0, bw0,
           Q1, bq1, W1, bw1, bias):
    fc_p = jnp.pad(feat_cat.astype(_i32),
                   ((0, 0), (0, NPR - N_SRC))).reshape(-1)
    ids_p = jnp.pad(ids.astype(_i32), (0, NPR - N_SRC))
    genre_p = jnp.pad(genre, ((0, NPR - N_SRC), (0, 0)))
    src0_p = jnp.pad(src0.astype(_i32), (0, E0P - src0.shape[0])).reshape(-1, CH)
    dst0_p = jnp.pad(dst0.astype(_i32), (0, E0P - dst0.shape[0])).reshape(-1, CH)
    w0_p = jnp.pad(w0, (0, E0P - w0.shape[0])).reshape(-1, CH)
    src1_p = jnp.pad(src1.astype(_i32), (0, E1P - src1.shape[0])).reshape(-1, CH)
    dst1_p = jnp.pad(dst1.astype(_i32), (0, E1P - dst1.shape[0])).reshape(-1, CH)
    w1_p = jnp.pad(w1, (0, E1P - w1.shape[0])).reshape(-1, CH)
    halfpad = PP // 2 - P
    ue = jnp.concatenate([jnp.pad(pos_edges[0].astype(_i32), (0, halfpad)),
                          jnp.pad(neg_edges[0].astype(_i32), (0, halfpad))])
    ve = jnp.concatenate([jnp.pad(pos_edges[1].astype(_i32), (0, halfpad)),
                          jnp.pad(neg_edges[1].astype(_i32), (0, halfpad))])
    emb_flat = emb_feats.reshape(9, 100, 16)
    bfc2 = bfc.reshape(1, D)
    bq02 = bq0.reshape(1, D)
    bw02 = bw0.reshape(1, D)
    bq12 = bq1.reshape(1, D)
    bw12 = bw1.reshape(1, D)

    t9, g_part = pl.pallas_call(
        _prep_body,
        out_shape=(jax.ShapeDtypeStruct((9, 100, D), _f32),
                   jax.ShapeDtypeStruct((NPR, D), _f32)),
    )(emb_flat, Wfc, genre_p, bfc2)
    tflat = t9.reshape(900, D)

    h = _gather_sum(tflat, id_table, g_part, fc_p, ids_p)

    n0 = pl.pallas_call(
        _dense1_body,
        out_shape=jax.ShapeDtypeStruct((NPR, D), _f32),
    )(h, Q0, bq02)

    agg0, ws0f = _edge_agg(n0, src0_p, dst0_p, w0_p, NP_MID, E0P, frac0=0.85)
    ws0 = ws0f.reshape(NC, NP_MID)

    h1, n1 = pl.pallas_call(
        functools.partial(_combine_body, n_dst=N_MID),
        out_shape=(jax.ShapeDtypeStruct((N_MID, D), _f32),
                   jax.ShapeDtypeStruct((N_MID, D), _f32)),
    )(agg0, ws0, h, W0, bw02, Q1, bq12)

    agg1, ws1f = _edge_agg(n1, src1_p, dst1_p, w1_p, NP_DST, E1P, frac0=0.85)
    ws1 = ws1f.reshape(NC, NP_DST)

    h_item = pl.pallas_call(
        _combine3_body,
        out_shape=jax.ShapeDtypeStruct((N_DST, D), _f32),
    )(agg1, ws1, h1, W1, bw12, h)

    ru, rv, bu, bv = _score(h_item, ue, ve, nid.astype(_i32), bias)

    loss_pad, auc2 = pl.pallas_call(
        _finish_body,
        out_shape=(jax.ShapeDtypeStruct((PP // 2,), _f32),
                   jax.ShapeDtypeStruct((1, 1), _f32)),
    )(ru, rv, bu, bv)

    return (loss_pad[:P], auc2.reshape(()))


# speedup vs baseline: 1.1133x; 1.1133x over previous
import functools

import jax
import jax.numpy as jnp
from jax import lax
from jax.experimental import pallas as pl
from jax.experimental.pallas import tpu as pltpu
from jax.experimental.pallas import tpu_sc as plsc

N_SRC, N_MID, N_DST = 10000, 5000, 2000
D = 128
VOCAB_ID = 100000
P = 2000

NC, NS = 2, 16
NW = NC * NS
CH = 128

NPR = 10240
NP_MID = 5120
NP_DST = 2048
E0P = 80 * 4096
E1P = 40 * 4096
PP = 4096

_f32 = jnp.float32
_i32 = jnp.int32


def _mesh():
    return plsc.VectorSubcoreMesh(core_axis_name="c", subcore_axis_name="s",
                                  num_cores=NC, num_subcores=NS)


def _zero_vec(ref, n_rows, n_cols):
    @pl.loop(0, n_rows)
    def _(r):
        for j in range(n_cols // 16):
            ref[r, pl.ds(j * 16, 16)] = jnp.zeros((16,), _f32)


def _acc_add(acc_v, buf):
    @pl.loop(0, CH)
    def _(r):
        for j in range(D // 16):
            sl = pl.ds(j * 16, 16)
            acc_v[r, sl] = acc_v[r, sl] + buf[r, sl]


def _gather_sum_body(tflat, idtab, g_part, fc, ids_in, h_out,
                     idx_a, idx_b, buf_a, buf_b, acc_v, sem_a, sem_b):
    c = lax.axis_index("c")
    s = lax.axis_index("s")
    cpc = NPR // CH // NC

    def chunk(off):
        pltpu.sync_copy(g_part.at[pl.ds(off, CH)], acc_v)
        pltpu.sync_copy(ids_in.at[pl.ds(off, CH)], idx_a)
        pltpu.async_copy(idtab.at[idx_a], buf_a, sem_a)
        for m in range(9):
            nidx, nbuf, nsem = (idx_b, buf_b, sem_b) if m % 2 == 0 else \
                               (idx_a, buf_a, sem_a)
            cbuf, csem = (buf_a, sem_a) if m % 2 == 0 else (buf_b, sem_b)
            pltpu.sync_copy(fc.at[pl.ds(m * NPR + off, CH)], nidx)
            for j in range(CH // 16):
                sl = pl.ds(j * 16, 16)
                nidx[sl] = nidx[sl] + (m * 100)
            pltpu.async_copy(tflat.at[nidx], nbuf, nsem)
            pltpu.make_async_copy(idtab.at[idx_a], cbuf, csem).wait()
            _acc_add(acc_v, cbuf)
        lbuf, lsem = (buf_b, sem_b) if 9 % 2 == 1 else (buf_a, sem_a)
        pltpu.make_async_copy(idtab.at[idx_a], lbuf, lsem).wait()
        _acc_add(acc_v, lbuf)
        pltpu.sync_copy(acc_v, h_out.at[pl.ds(off, CH)])

    del cpc
    for kk in range(3):
        @pl.when(c == 0)
        def _():
            chunk((s + NS * kk) * CH)

    for kk in range(2):
        @pl.when(c == 1)
        def _():
            chunk((3 * NS + kk * NS + s) * CH)


def _gather_sum(tflat, idtab, g_part, fc, ids_in):
    kfn = pl.kernel(
        _gather_sum_body,
        out_type=jax.ShapeDtypeStruct((NPR, D), _f32),
        mesh=_mesh(),
        scratch_types=[
            pltpu.VMEM((CH,), _i32),
            pltpu.VMEM((CH,), _i32),
            pltpu.VMEM((CH, D), _f32),
            pltpu.VMEM((CH, D), _f32),
            pltpu.VMEM((CH, D), _f32),
            pltpu.SemaphoreType.DMA,
            pltpu.SemaphoreType.DMA,
        ],
    )
    return kfn(tflat, idtab, g_part, fc, ids_in)


def _edge_agg_body(n_hbm, src, dst, w, agg_out, ws_out,
                   si_v, di_v, w_v, g_a, g_b, z1_v,
                   sem_a, sem_b, sem_c, sem_d, agg_s, ws_s,
                   *, n_pad, e_pad, ca, cb_n):
    c = lax.axis_index("c")
    s = lax.axis_index("s")
    rows_per_tile = n_pad // NS

    _zero_vec(g_a, CH, D)
    for j in range(CH // 16):
        z1_v[pl.ds(j * 16, 16)] = jnp.zeros((16,), _f32)
    r0 = s * rows_per_tile
    off = 0
    while off < rows_per_tile:
        n = min(CH, rows_per_tile - off)
        pltpu.sync_copy(g_a.at[pl.ds(0, n)], agg_s.at[pl.ds(r0 + off, n)])
        pltpu.sync_copy(z1_v.at[pl.ds(0, n)], ws_s.at[pl.ds(r0 + off, n)])
        off += n
    plsc.subcore_barrier()

    def scale(buf, k):
        @pl.loop(0, CH // 16)
        def _(g):
            wv = w_v[k, pl.ds(g * 16, 16)]
            for l in range(16):
                spl = jnp.broadcast_to(wv[l:l + 1], (16,))
                e = g * 16 + l
                for j in range(D // 16):
                    sl = pl.ds(j * 16, 16)
                    buf[e, sl] = buf[e, sl] * spl

    def emit(buf, k, sem):
        pltpu.async_copy(buf, agg_s.at[di_v.at[k]], sem, add=True)
        pltpu.async_copy(w_v.at[k], ws_s.at[di_v.at[k]], sem, add=True)

    def drain(buf, sem):
        pltpu.make_async_copy(buf, agg_s.at[di_v.at[0]], sem).wait()
        pltpu.make_async_copy(w_v.at[0], ws_s.at[di_v.at[0]], sem).wait()

    def run(n_chunks, cb):
        pltpu.sync_copy(src.at[pl.ds(cb, n_chunks)],
                        si_v.at[pl.ds(0, n_chunks)])
        pltpu.sync_copy(dst.at[pl.ds(cb, n_chunks)],
                        di_v.at[pl.ds(0, n_chunks)])
        pltpu.sync_copy(w.at[pl.ds(cb, n_chunks)],
                        w_v.at[pl.ds(0, n_chunks)])

        pltpu.async_copy(n_hbm.at[si_v.at[0]], g_a, sem_a)

        @pl.loop(0, n_chunks // 2)
        def _(p):
            k0 = 2 * p
            pltpu.make_async_copy(n_hbm.at[si_v.at[0]], g_a, sem_a).wait()
            pltpu.async_copy(n_hbm.at[si_v.at[k0 + 1]], g_b, sem_b)
            scale(g_a, k0)
            emit(g_a, k0, sem_c)

            pltpu.make_async_copy(n_hbm.at[si_v.at[0]], g_b, sem_b).wait()
            drain(g_a, sem_c)

            @pl.when(k0 + 2 < n_chunks)
            def _():
                pltpu.async_copy(n_hbm.at[si_v.at[k0 + 2]], g_a, sem_a)
            scale(g_b, k0 + 1)
            emit(g_b, k0 + 1, sem_d)
            drain(g_b, sem_d)

    @pl.when(c == 0)
    def _():
        run(ca, s * ca)

    if cb_n > 0:
        @pl.when(c == 1)
        def _():
            run(cb_n, NS * ca + s * cb_n)

    plsc.subcore_barrier()
    off = 0
    while off < rows_per_tile:
        n = min(CH, rows_per_tile - off)
        pltpu.sync_copy(agg_s.at[pl.ds(r0 + off, n)], g_a.at[pl.ds(0, n)])
        pltpu.sync_copy(g_a.at[pl.ds(0, n)], agg_out.at[c, pl.ds(r0 + off, n)])
        pltpu.sync_copy(ws_s.at[pl.ds(r0 + off, n)], z1_v.at[pl.ds(0, n)])
        pltpu.sync_copy(z1_v.at[pl.ds(0, n)],
                        ws_out.at[pl.ds(c * n_pad + r0 + off, n)])
        off += n


def _edge_agg(n_hbm, src2d, dst2d, w2d, n_pad, e_pad, frac0=0.5):
    pair_chunks = e_pad // NS // CH
    ca = max(8, 8 * int(round(pair_chunks * frac0 / 8)))
    ca = min(ca, pair_chunks)
    cb_n = pair_chunks - ca
    body = functools.partial(_edge_agg_body, n_pad=n_pad, e_pad=e_pad,
                             ca=ca, cb_n=cb_n)
    kfn = pl.kernel(
        body,
        out_type=(jax.ShapeDtypeStruct((NC, n_pad, D), _f32),
                  jax.ShapeDtypeStruct((NC * n_pad,), _f32)),
        mesh=_mesh(),
        scratch_types=[
            pltpu.VMEM((max(ca, cb_n), CH), _i32),
            pltpu.VMEM((max(ca, cb_n), CH), _i32),
            pltpu.VMEM((max(ca, cb_n), CH), _f32),
            pltpu.VMEM((CH, D), _f32),
            pltpu.VMEM((CH, D), _f32),
            pltpu.VMEM((CH,), _f32),
            pltpu.SemaphoreType.DMA,
            pltpu.SemaphoreType.DMA,
            pltpu.SemaphoreType.DMA,
            pltpu.SemaphoreType.DMA,
            pltpu.VMEM_SHARED((n_pad, D), _f32),
            pltpu.VMEM_SHARED((n_pad,), _f32),
        ],
    )
    return kfn(n_hbm, src2d, dst2d, w2d)


def _score_body(h_item, ue, ve, nid_h, bias_h, ru_o, rv_o, bu_o, bv_o,
                ui_v, vi_v, ru_v, rv_v, nu_v, bu_v, sem):
    c = lax.axis_index("c")
    s = lax.axis_index("s")
    wid = c * NS + s
    base = wid * (PP // NW)
    pltpu.sync_copy(ue.at[pl.ds(base, CH)], ui_v)
    pltpu.sync_copy(ve.at[pl.ds(base, CH)], vi_v)
    pltpu.async_copy(h_item.at[ui_v], ru_v, sem).wait()
    pltpu.sync_copy(ru_v, ru_o.at[pl.ds(base, CH)])
    pltpu.async_copy(h_item.at[vi_v], rv_v, sem).wait()
    pltpu.sync_copy(rv_v, rv_o.at[pl.ds(base, CH)])
    pltpu.async_copy(nid_h.at[ui_v], nu_v, sem).wait()
    pltpu.async_copy(bias_h.at[nu_v], bu_v, sem).wait()
    pltpu.sync_copy(bu_v, bu_o.at[pl.ds(base, CH)])
    pltpu.async_copy(nid_h.at[vi_v], nu_v, sem).wait()
    pltpu.async_copy(bias_h.at[nu_v], bu_v, sem).wait()
    pltpu.sync_copy(bu_v, bv_o.at[pl.ds(base, CH)])


def _score(h_item, ue, ve, nid_h, bias_h):
    kfn = pl.kernel(
        _score_body,
        out_type=(jax.ShapeDtypeStruct((PP, D), _f32),
                  jax.ShapeDtypeStruct((PP, D), _f32),
                  jax.ShapeDtypeStruct((PP,), _f32),
                  jax.ShapeDtypeStruct((PP,), _f32)),
        mesh=_mesh(),
        scratch_types=[
            pltpu.VMEM((CH,), _i32),
            pltpu.VMEM((CH,), _i32),
            pltpu.VMEM((CH, D), _f32),
            pltpu.VMEM((CH, D), _f32),
            pltpu.VMEM((CH,), _i32),
            pltpu.VMEM((CH,), _f32),
            pltpu.SemaphoreType.DMA,
        ],
    )
    return kfn(h_item, ue, ve, nid_h, bias_h)


def _dot_t(x, w):
    return lax.dot_general(x, w, (((1,), (1,)), ((), ())),
                           preferred_element_type=_f32)


def _prep_body(emb_ref, wfc_ref, genre_ref, bfc_ref, t_ref, g_ref):
    for m in range(9):
        wm = wfc_ref[:, 16 * m:16 * (m + 1)]
        t_ref[m] = _dot_t(emb_ref[m], wm)
    wg = wfc_ref[:, 144:164]
    g_ref[...] = _dot_t(genre_ref[...], wg) + bfc_ref[...]


def _dense1_body(h_ref, q_ref, bq_ref, n_ref):
    n_ref[...] = jnp.maximum(_dot_t(h_ref[...], q_ref[...]) + bq_ref[...], 0.0)


def _combine_body(agg_ref, ws_ref, hd_ref, w_ref, bw_ref, q_ref, bq_ref,
                  h1_ref, n1_ref, *, n_dst):
    p = agg_ref[0, pl.ds(0, n_dst)] + agg_ref[1, pl.ds(0, n_dst)]
    ws = jnp.clip(ws_ref[0, pl.ds(0, n_dst)] + ws_ref[1, pl.ds(0, n_dst)],
                  1.0, None)
    x = p / ws[:, None]
    hd = hd_ref[pl.ds(0, n_dst)]
    wa = w_ref[:, 0:D]
    wb = w_ref[:, D:2 * D]
    z = jnp.maximum(_dot_t(x, wa) + _dot_t(hd, wb) + bw_ref[...], 0.0)
    zn = jnp.sqrt(jnp.sum(z * z, axis=1, keepdims=True))
    zn = jnp.where(zn == 0.0, 1.0, zn)
    h1 = z / zn
    h1_ref[...] = h1
    if n1_ref is not None:
        n1_ref[...] = jnp.maximum(_dot_t(h1, q_ref[...]) + bq_ref[...], 0.0)


def _combine3_body(agg_ref, ws_ref, hd_ref, w_ref, bw_ref, hp_ref, hi_ref):
    p = agg_ref[0, pl.ds(0, N_DST)] + agg_ref[1, pl.ds(0, N_DST)]
    ws = jnp.clip(ws_ref[0, pl.ds(0, N_DST)] + ws_ref[1, pl.ds(0, N_DST)],
                  1.0, None)
    x = p / ws[:, None]
    hd = hd_ref[pl.ds(0, N_DST)]
    wa = w_ref[:, 0:D]
    wb = w_ref[:, D:2 * D]
    z = jnp.maximum(_dot_t(x, wa) + _dot_t(hd, wb) + bw_ref[...], 0.0)
    zn = jnp.sqrt(jnp.sum(z * z, axis=1, keepdims=True))
    zn = jnp.where(zn == 0.0, 1.0, zn)
    hi_ref[...] = hp_ref[pl.ds(0, N_DST)] + z / zn


def _finish_body(ru_ref, rv_ref, bu_ref, bv_ref, loss_ref, auc_ref):
    s = (jnp.sum(ru_ref[...] * rv_ref[...], axis=1)
         + bu_ref[...] + bv_ref[...])
    half = PP // 2
    pos = s[0:half]
    neg = s[half:PP]
    loss_ref[...] = jnp.maximum(neg - pos + 1.0, 0.0)
    valid = lax.broadcasted_iota(_i32, (half,), 0) < P
    pv = jnp.where(valid, pos, -jnp.inf)
    nv = jnp.where(valid, neg, jnp.inf)
    gt = jnp.sum((pv[:, None] > nv[None, :]).astype(_f32))
    eq = jnp.sum((pv[:, None] == nv[None, :]).astype(_f32))
    auc_ref[...] = jnp.broadcast_to((gt + 0.5 * eq) / (float(P) * float(P)),
                                    (1, 1))


def kernel(feat_cat, ids, genre, src0, dst0, w0, src1, dst1, w1, pos_edges,
           neg_edges, nid, emb_feats, id_table, Wfc, bfc, Q0, bq0, W0, bw0,
           Q1, bq1, W1, bw1, bias):
    fc_p = jnp.pad(feat_cat.astype(_i32),
                   ((0, 0), (0, NPR - N_SRC))).reshape(-1)
    ids_p = jnp.pad(ids.astype(_i32), (0, NPR - N_SRC))
    genre_p = jnp.pad(genre, ((0, NPR - N_SRC), (0, 0)))
    src0_p = jnp.pad(src0.astype(_i32), (0, E0P - src0.shape[0])).reshape(-1, CH)
    dst0_p = jnp.pad(dst0.astype(_i32), (0, E0P - dst0.shape[0])).reshape(-1, CH)
    w0_p = jnp.pad(w0, (0, E0P - w0.shape[0])).reshape(-1, CH)
    src1_p = jnp.pad(src1.astype(_i32), (0, E1P - src1.shape[0])).reshape(-1, CH)
    dst1_p = jnp.pad(dst1.astype(_i32), (0, E1P - dst1.shape[0])).reshape(-1, CH)
    w1_p = jnp.pad(w1, (0, E1P - w1.shape[0])).reshape(-1, CH)
    halfpad = PP // 2 - P
    ue = jnp.concatenate([jnp.pad(pos_edges[0].astype(_i32), (0, halfpad)),
                          jnp.pad(neg_edges[0].astype(_i32), (0, halfpad))])
    ve = jnp.concatenate([jnp.pad(pos_edges[1].astype(_i32), (0, halfpad)),
                          jnp.pad(neg_edges[1].astype(_i32), (0, halfpad))])
    emb_flat = emb_feats.reshape(9, 100, 16)
    bfc2 = bfc.reshape(1, D)
    bq02 = bq0.reshape(1, D)
    bw02 = bw0.reshape(1, D)
    bq12 = bq1.reshape(1, D)
    bw12 = bw1.reshape(1, D)

    t9, g_part = pl.pallas_call(
        _prep_body,
        out_shape=(jax.ShapeDtypeStruct((9, 100, D), _f32),
                   jax.ShapeDtypeStruct((NPR, D), _f32)),
    )(emb_flat, Wfc, genre_p, bfc2)
    tflat = t9.reshape(900, D)

    h = _gather_sum(tflat, id_table, g_part, fc_p, ids_p)

    n0 = pl.pallas_call(
        _dense1_body,
        out_shape=jax.ShapeDtypeStruct((NPR, D), _f32),
    )(h, Q0, bq02)

    agg0, ws0f = _edge_agg(n0, src0_p, dst0_p, w0_p, NP_MID, E0P, frac0=0.9)
    ws0 = ws0f.reshape(NC, NP_MID)

    h1, n1 = pl.pallas_call(
        functools.partial(_combine_body, n_dst=N_MID),
        out_shape=(jax.ShapeDtypeStruct((N_MID, D), _f32),
                   jax.ShapeDtypeStruct((N_MID, D), _f32)),
    )(agg0, ws0, h, W0, bw02, Q1, bq12)

    agg1, ws1f = _edge_agg(n1, src1_p, dst1_p, w1_p, NP_DST, E1P, frac0=0.9)
    ws1 = ws1f.reshape(NC, NP_DST)

    h_item = pl.pallas_call(
        _combine3_body,
        out_shape=jax.ShapeDtypeStruct((N_DST, D), _f32),
    )(agg1, ws1, h1, W1, bw12, h)

    ru, rv, bu, bv = _score(h_item, ue, ve, nid.astype(_i32), bias)

    loss_pad, auc2 = pl.pallas_call(
        _finish_body,
        out_shape=(jax.ShapeDtypeStruct((PP // 2,), _f32),
                   jax.ShapeDtypeStruct((1, 1), _f32)),
    )(ru, rv, bu, bv)

    return (loss_pad[:P], auc2.reshape(()))
